# Initial kernel scaffold; baseline (speedup 1.0000x reference)
import jax
import jax.numpy as jnp
from jax.experimental import pallas as pl


def kernel(x, W_experts, b_experts, W_gate, b_gate):
    raise NotImplementedError("write your pallas kernel here")


# speedup vs baseline: 3.2034x; 3.2034x over previous
import functools

import jax
import jax.numpy as jnp
from jax.experimental import pallas as pl
from jax.experimental.pallas import tpu as pltpu

_LAMBDA = 1.0
_NEG_INF = float("-inf")
_BT = 1024
_NTB = 4096 // _BT


def _moe_dense_kernel(x_ref, wg_ref, bg_ref, w_ref, b_ref, out_ref, cv_ref,
                      p_scratch, load_scratch):
    tb = pl.program_id(0)
    e = pl.program_id(1)

    @pl.when(e == 0)
    def _gate():
        xv = x_ref[...]
        logits = jax.lax.dot_general(
            xv, wg_ref[...], (((1,), (1,)), ((), ())),
            preferred_element_type=jnp.float32) + bg_ref[...]
        idx8 = jax.lax.broadcasted_iota(jnp.int32, (_BT, 8), 1)
        m1 = jnp.max(logits, axis=1, keepdims=True)
        i1 = jnp.min(jnp.where(logits == m1, idx8, 8), axis=1, keepdims=True)
        sel1 = idx8 == i1
        masked = jnp.where(sel1, _NEG_INF, logits)
        m2 = jnp.max(masked, axis=1, keepdims=True)
        i2 = jnp.min(jnp.where(masked == m2, idx8, 8), axis=1, keepdims=True)
        sel2 = idx8 == i2
        e2 = jnp.exp(m2 - m1)
        z = 1.0 + e2
        p1 = 1.0 / z
        p2 = e2 / z
        pfull = jnp.where(sel1, p1, jnp.where(sel2, p2, 0.0))
        p_scratch[...] = pfull
        blk_load = jnp.sum(pfull, axis=0, keepdims=True)

        @pl.when(tb == 0)
        def _():
            load_scratch[...] = blk_load

        @pl.when(tb != 0)
        def _():
            load_scratch[...] += blk_load

        @pl.when(tb == _NTB - 1)
        def _():
            load = load_scratch[...]
            mean = jnp.sum(load) / 8.0
            var = jnp.sum((load - mean) ** 2) / 7.0
            cv = jnp.sqrt(var) / mean
            cv_ref[...] = jnp.full((8, 128), cv, jnp.float32)

    pe = jnp.sum(
        jnp.where(
            jax.lax.broadcasted_iota(jnp.int32, (_BT, 8), 1) == e,
            p_scratch[...], 0.0),
        axis=1, keepdims=True)
    xb = x_ref[...].astype(jnp.bfloat16)
    wb = w_ref[0].astype(jnp.bfloat16)
    y = jax.lax.dot_general(
        xb, wb, (((1,), (1,)), ((), ())),
        preferred_element_type=jnp.float32) + b_ref[0]
    contrib = pe * y

    @pl.when(e == 0)
    def _init():
        out_ref[...] = contrib

    @pl.when(e != 0)
    def _acc():
        out_ref[...] += contrib


def kernel(x, W_experts, b_experts, W_gate, b_gate):
    out, cvb = pl.pallas_call(
        _moe_dense_kernel,
        grid=(_NTB, 8),
        in_specs=[
            pl.BlockSpec((_BT, 1024), lambda tb, e: (tb, 0)),
            pl.BlockSpec((8, 1024), lambda tb, e: (0, 0)),
            pl.BlockSpec((1, 8), lambda tb, e: (0, 0)),
            pl.BlockSpec((1, 1024, 1024), lambda tb, e: (e, 0, 0)),
            pl.BlockSpec((1, 1, 1024), lambda tb, e: (e, 0, 0)),
        ],
        out_specs=[
            pl.BlockSpec((_BT, 1024), lambda tb, e: (tb, 0)),
            pl.BlockSpec((8, 128), lambda tb, e: (0, 0)),
        ],
        out_shape=[
            jax.ShapeDtypeStruct((4096, 1024), jnp.float32),
            jax.ShapeDtypeStruct((8, 128), jnp.float32),
        ],
        scratch_shapes=[
            pltpu.VMEM((_BT, 8), jnp.float32),
            pltpu.VMEM((1, 8), jnp.float32),
        ],
    )(x, W_gate, b_gate.reshape(1, 8), W_experts,
      b_experts.reshape(8, 1, 1024))
    return (out, _LAMBDA * cvb[0, 0])


# speedup vs baseline: 3.3021x; 1.0308x over previous
import functools

import jax
import jax.numpy as jnp
from jax.experimental import pallas as pl
from jax.experimental.pallas import tpu as pltpu

_LAMBDA = 1.0
_NEG_INF = float("-inf")
_BT = 2048
_NTB = 4096 // _BT


def _moe_dense_kernel(x_ref, wg_ref, bg_ref, w_ref, b_ref, out_ref, cv_ref,
                      p_scratch, load_scratch):
    tb = pl.program_id(0)
    e = pl.program_id(1)

    @pl.when(e == 0)
    def _gate():
        xv = x_ref[...]
        logits = jax.lax.dot_general(
            xv, wg_ref[...], (((1,), (1,)), ((), ())),
            preferred_element_type=jnp.float32) + bg_ref[...]
        idx8 = jax.lax.broadcasted_iota(jnp.int32, (_BT, 8), 1)
        m1 = jnp.max(logits, axis=1, keepdims=True)
        i1 = jnp.min(jnp.where(logits == m1, idx8, 8), axis=1, keepdims=True)
        sel1 = idx8 == i1
        masked = jnp.where(sel1, _NEG_INF, logits)
        m2 = jnp.max(masked, axis=1, keepdims=True)
        i2 = jnp.min(jnp.where(masked == m2, idx8, 8), axis=1, keepdims=True)
        sel2 = idx8 == i2
        e2 = jnp.exp(m2 - m1)
        z = 1.0 + e2
        p1 = 1.0 / z
        p2 = e2 / z
        pfull = jnp.where(sel1, p1, jnp.where(sel2, p2, 0.0))
        p_scratch[...] = pfull
        blk_load = jnp.sum(pfull, axis=0, keepdims=True)

        @pl.when(tb == 0)
        def _():
            load_scratch[...] = blk_load

        @pl.when(tb != 0)
        def _():
            load_scratch[...] += blk_load

        @pl.when(tb == _NTB - 1)
        def _():
            load = load_scratch[...]
            mean = jnp.sum(load) / 8.0
            var = jnp.sum((load - mean) ** 2) / 7.0
            cv = jnp.sqrt(var) / mean
            cv_ref[...] = jnp.full((8, 128), cv, jnp.float32)

    pe = jnp.sum(
        jnp.where(
            jax.lax.broadcasted_iota(jnp.int32, (_BT, 8), 1) == e,
            p_scratch[...], 0.0),
        axis=1, keepdims=True)
    xb = x_ref[...].astype(jnp.bfloat16)
    wb = w_ref[0].astype(jnp.bfloat16)
    y = jax.lax.dot_general(
        xb, wb, (((1,), (1,)), ((), ())),
        preferred_element_type=jnp.float32) + b_ref[0]
    contrib = pe * y

    @pl.when(e == 0)
    def _init():
        out_ref[...] = contrib

    @pl.when(e != 0)
    def _acc():
        out_ref[...] += contrib


def kernel(x, W_experts, b_experts, W_gate, b_gate):
    out, cvb = pl.pallas_call(
        _moe_dense_kernel,
        grid=(_NTB, 8),
        in_specs=[
            pl.BlockSpec((_BT, 1024), lambda tb, e: (tb, 0)),
            pl.BlockSpec((8, 1024), lambda tb, e: (0, 0)),
            pl.BlockSpec((1, 8), lambda tb, e: (0, 0)),
            pl.BlockSpec((1, 1024, 1024), lambda tb, e: (e, 0, 0)),
            pl.BlockSpec((1, 1, 1024), lambda tb, e: (e, 0, 0)),
        ],
        out_specs=[
            pl.BlockSpec((_BT, 1024), lambda tb, e: (tb, 0)),
            pl.BlockSpec((8, 128), lambda tb, e: (0, 0)),
        ],
        out_shape=[
            jax.ShapeDtypeStruct((4096, 1024), jnp.float32),
            jax.ShapeDtypeStruct((8, 128), jnp.float32),
        ],
        scratch_shapes=[
            pltpu.VMEM((_BT, 8), jnp.float32),
            pltpu.VMEM((1, 8), jnp.float32),
        ],
    )(x, W_gate, b_gate.reshape(1, 8), W_experts,
      b_experts.reshape(8, 1, 1024))
    return (out, _LAMBDA * cvb[0, 0])
